# baseline (speedup 1.0000x reference)
import functools

import jax
import jax.numpy as jnp
from jax import lax
from jax.experimental import pallas as pl
from jax.experimental.pallas import tpu as pltpu
from jax.experimental.pallas import tpu_sc as plsc

VOCAB = 1_000_000
HIDDEN = 64
BATCH = 16384
SEQ = 50

LANES = 16
NC = 2
NS = 16
NW = NC * NS
ROWS_PER_W = BATCH // NW
CHUNK = 16
NCHUNK = ROWS_PER_W // CHUNK
GATHER_N = CHUNK * SEQ
SUB = 128
NSUB = (GATHER_N + SUB - 1) // SUB
TAIL = GATHER_N - (NSUB - 1) * SUB
GATHER_PAD = NSUB * SUB

_mesh = plsc.VectorSubcoreMesh(core_axis_name="c", subcore_axis_name="s")


@functools.partial(
    pl.kernel,
    out_type=jax.ShapeDtypeStruct((BATCH, HIDDEN), jnp.float32),
    mesh=_mesh,
    compiler_params=pltpu.CompilerParams(
        needs_layout_passes=False, use_tc_tiling_on_sc=False),
    scratch_types=[
        pltpu.VMEM((NSUB, SUB), jnp.int32),
        pltpu.VMEM((GATHER_N,), jnp.int32),
        pltpu.VMEM((GATHER_PAD, HIDDEN), jnp.float32),
        pltpu.VMEM((CHUNK, HIDDEN), jnp.float32),
        pltpu.VMEM((LANES,), jnp.float32),
        pltpu.SemaphoreType.DMA,
    ],
)
def _header_embed(tok_hbm, table_hbm, out_hbm, idx_v, cnt_v, rows_v, out_v,
                  rcp_v, sem):
    wid = lax.axis_index("s") * NC + lax.axis_index("c")
    wbase = wid * ROWS_PER_W

    zeros16 = jnp.zeros((LANES,), jnp.int32)
    for s in range(TAIL // LANES, SUB // LANES):
        idx_v[NSUB - 1, pl.ds(s * LANES, LANES)] = zeros16

    lane = lax.iota(jnp.int32, LANES)

    def chunk_body(g, carry):
        base = wbase + g * CHUNK
        tok0 = base * SEQ

        for i in range(NSUB - 1):
            pltpu.sync_copy(tok_hbm.at[pl.ds(tok0 + i * SUB, SUB)],
                            idx_v.at[i])
        pltpu.sync_copy(tok_hbm.at[pl.ds(tok0 + (NSUB - 1) * SUB, TAIL)],
                        idx_v.at[NSUB - 1, pl.ds(0, TAIL)])
        pltpu.sync_copy(tok_hbm.at[pl.ds(tok0, GATHER_N)], cnt_v)

        copies = [
            pltpu.async_copy(table_hbm.at[idx_v.at[i]],
                             rows_v.at[pl.ds(i * SUB, SUB)], sem)
            for i in range(NSUB)
        ]
        for c in copies:
            c.wait()

        def cnt_body(j, tot):
            v = plsc.load_gather(cnt_v, [lane * SEQ + j])
            return tot + jnp.where(v != 0, 1, 0)

        total = lax.fori_loop(0, SEQ, cnt_body, zeros16)
        rcp_v[...] = 1.0 / total.astype(jnp.float32)

        def row_body(r, carry2):
            rbase = r * SEQ
            rcp = plsc.load_gather(rcp_v, [jnp.broadcast_to(r, (LANES,))])

            def tok_body(j, acc):
                p = rbase + j
                return tuple(
                    acc[k] + rows_v[p, pl.ds(k * LANES, LANES)]
                    for k in range(HIDDEN // LANES)
                )

            acc = lax.fori_loop(
                0, SEQ, tok_body,
                tuple(jnp.zeros((LANES,), jnp.float32)
                      for _ in range(HIDDEN // LANES)))
            for k in range(HIDDEN // LANES):
                out_v[r, pl.ds(k * LANES, LANES)] = acc[k] * rcp
            return carry2

        lax.fori_loop(0, CHUNK, row_body, 0)
        pltpu.sync_copy(out_v, out_hbm.at[pl.ds(base, CHUNK)])
        return carry

    lax.fori_loop(0, NCHUNK, chunk_body, 0)


def kernel(header_tok, table):
    tok_flat = header_tok.reshape(-1)
    return _header_embed(tok_flat, table)


# speedup vs baseline: 3.6073x; 3.6073x over previous
import functools

import jax
import jax.numpy as jnp
from jax import lax
from jax.experimental import pallas as pl
from jax.experimental.pallas import tpu as pltpu
from jax.experimental.pallas import tpu_sc as plsc

VOCAB = 1_000_000
HIDDEN = 64
BATCH = 16384
SEQ = 50

LANES = 16
NC = 2
NS = 16
NW = NC * NS
ROWS_PER_W = BATCH // NW
CHUNK = 16
NCHUNK = ROWS_PER_W // CHUNK
GATHER_N = CHUNK * SEQ
NVEC = HIDDEN // LANES

_mesh = plsc.VectorSubcoreMesh(core_axis_name="c", subcore_axis_name="s")


@functools.partial(
    pl.kernel,
    out_type=jax.ShapeDtypeStruct((BATCH, HIDDEN), jnp.float32),
    mesh=_mesh,
    compiler_params=pltpu.CompilerParams(
        needs_layout_passes=False, use_tc_tiling_on_sc=False),
    scratch_types=[
        [pltpu.VMEM((GATHER_N,), jnp.int32) for _ in range(2)],
        [pltpu.VMEM((GATHER_N, HIDDEN), jnp.float32) for _ in range(2)],
        [pltpu.VMEM((CHUNK, HIDDEN), jnp.float32) for _ in range(2)],
        pltpu.VMEM((LANES,), jnp.float32),
        [pltpu.SemaphoreType.DMA for _ in range(2)],
        [pltpu.SemaphoreType.DMA for _ in range(2)],
    ],
)
def _header_embed(tok_hbm, table_hbm, out_hbm, tok_v, rows_v, out_v, rcp_v,
                  gsem, osem):
    wid = lax.axis_index("s") * NC + lax.axis_index("c")
    wbase = wid * ROWS_PER_W

    lane = lax.iota(jnp.int32, LANES)
    zeros16 = jnp.zeros((LANES,), jnp.int32)

    def stage_and_fire(g, a):
        tok0 = (wbase + g * CHUNK) * SEQ
        pltpu.sync_copy(tok_hbm.at[pl.ds(tok0, GATHER_N)], tok_v[a])
        pltpu.async_copy(table_hbm.at[tok_v[a]], rows_v[a], gsem[a])

    def compute(g, a):
        base = wbase + g * CHUNK
        tok = tok_v[a]
        rows = rows_v[a]

        def cnt_body(j, tot):
            v = plsc.load_gather(tok, [lane * SEQ + j])
            return tot + jnp.where(v != 0, 1, 0)

        total = lax.fori_loop(0, SEQ, cnt_body, zeros16, unroll=5)
        rcp_v[...] = 1.0 / total.astype(jnp.float32)

        def row_body(r, carry):
            rbase = r * SEQ
            rcp = plsc.load_gather(rcp_v, [jnp.broadcast_to(r, (LANES,))])

            def tok_body(j, acc):
                p = rbase + j
                return tuple(
                    acc[k] + rows[p, pl.ds(k * LANES, LANES)]
                    for k in range(NVEC)
                )

            acc = lax.fori_loop(
                0, SEQ, tok_body,
                tuple(jnp.zeros((LANES,), jnp.float32) for _ in range(NVEC)),
                unroll=5)
            for k in range(NVEC):
                out_v[a][r, pl.ds(k * LANES, LANES)] = acc[k] * rcp
            return carry

        lax.fori_loop(0, CHUNK, row_body, 0)
        pltpu.async_copy(out_v[a], out_hbm.at[pl.ds(base, CHUNK)], osem[a])

    stage_and_fire(0, 0)

    def loop_body(i, carry):
        for half in range(2):
            g = 2 * i + half
            a = half
            b = 1 - half

            @pl.when(g + 1 < NCHUNK)
            def _():
                stage_and_fire(g + 1, b)

            pltpu.make_async_copy(table_hbm.at[tok_v[a]], rows_v[a],
                                  gsem[a]).wait()

            @pl.when(g >= 2)
            def _():
                pltpu.make_async_copy(
                    out_v[a], out_hbm.at[pl.ds(0, CHUNK)], osem[a]).wait()

            compute(g, a)
        return carry

    lax.fori_loop(0, NCHUNK // 2, loop_body, 0)
    for a in range(2):
        pltpu.make_async_copy(out_v[a], out_hbm.at[pl.ds(0, CHUNK)],
                              osem[a]).wait()


def kernel(header_tok, table):
    tok_flat = header_tok.reshape(-1)
    return _header_embed(tok_flat, table)


# speedup vs baseline: 3.6558x; 1.0134x over previous
import functools

import jax
import jax.numpy as jnp
from jax import lax
from jax.experimental import pallas as pl
from jax.experimental.pallas import tpu as pltpu
from jax.experimental.pallas import tpu_sc as plsc

VOCAB = 1_000_000
HIDDEN = 64
BATCH = 16384
SEQ = 50

LANES = 16
NC = 2
NS = 16
NW = NC * NS
ROWS_PER_W = BATCH // NW
CHUNK = 16
NCHUNK = ROWS_PER_W // CHUNK
GATHER_N = CHUNK * SEQ
NVEC = HIDDEN // LANES

_mesh = plsc.VectorSubcoreMesh(core_axis_name="c", subcore_axis_name="s")


@functools.partial(
    pl.kernel,
    out_type=jax.ShapeDtypeStruct((BATCH, HIDDEN), jnp.float32),
    mesh=_mesh,
    compiler_params=pltpu.CompilerParams(
        needs_layout_passes=False, use_tc_tiling_on_sc=False),
    scratch_types=[
        [pltpu.VMEM((GATHER_N,), jnp.int32) for _ in range(2)],
        [pltpu.VMEM((GATHER_N, HIDDEN), jnp.float32) for _ in range(2)],
        [pltpu.VMEM((CHUNK, HIDDEN), jnp.float32) for _ in range(2)],
        pltpu.VMEM((LANES,), jnp.float32),
        [pltpu.SemaphoreType.DMA for _ in range(2)],
        [pltpu.SemaphoreType.DMA for _ in range(2)],
        [pltpu.SemaphoreType.DMA for _ in range(2)],
    ],
)
def _header_embed(tok_hbm, table_hbm, out_hbm, tok_v, rows_v, out_v, rcp_v,
                  gsem, osem, tsem):
    wid = lax.axis_index("s") * NC + lax.axis_index("c")
    wbase = wid * ROWS_PER_W

    lane = lax.iota(jnp.int32, LANES)
    zeros16 = jnp.zeros((LANES,), jnp.int32)

    def stage_tok(g, a):
        tok0 = (wbase + g * CHUNK) * SEQ
        pltpu.async_copy(tok_hbm.at[pl.ds(tok0, GATHER_N)], tok_v[a], tsem[a])

    def wait_tok(g, a):
        tok0 = (wbase + g * CHUNK) * SEQ
        pltpu.make_async_copy(tok_hbm.at[pl.ds(tok0, GATHER_N)], tok_v[a],
                              tsem[a]).wait()

    def counts(a):
        tok = tok_v[a]

        def cnt_body(j, tot):
            v = plsc.load_gather(tok, [lane * SEQ + j])
            return tot + jnp.where(v != 0, 1, 0)

        total = lax.fori_loop(0, SEQ, cnt_body, zeros16, unroll=5)
        rcp_v[...] = 1.0 / total.astype(jnp.float32)

    def accumulate(g, a):
        base = wbase + g * CHUNK
        rows = rows_v[a]

        def row_body(r, carry):
            rbase = r * SEQ
            rcpb = plsc.load_gather(rcp_v, [jnp.broadcast_to(r, (LANES,))])

            def tok_body(j, acc):
                p = rbase + j
                return tuple(
                    acc[k] + rows[p, pl.ds(k * LANES, LANES)]
                    for k in range(NVEC)
                )

            acc = lax.fori_loop(
                0, SEQ, tok_body,
                tuple(jnp.zeros((LANES,), jnp.float32) for _ in range(NVEC)),
                unroll=5)
            for k in range(NVEC):
                out_v[a][r, pl.ds(k * LANES, LANES)] = acc[k] * rcpb
            return carry

        lax.fori_loop(0, CHUNK, row_body, 0)
        pltpu.async_copy(out_v[a], out_hbm.at[pl.ds(base, CHUNK)], osem[a])

    stage_tok(0, 0)
    wait_tok(0, 0)
    pltpu.async_copy(table_hbm.at[tok_v[0]], rows_v[0], gsem[0])
    stage_tok(1, 1)

    def loop_body(i, carry):
        for half in range(2):
            g = 2 * i + half
            a = half
            b = 1 - half

            pltpu.make_async_copy(table_hbm.at[tok_v[a]], rows_v[a],
                                  gsem[a]).wait()

            @pl.when(g + 1 < NCHUNK)
            def _():
                wait_tok(g + 1, b)
                pltpu.async_copy(table_hbm.at[tok_v[b]], rows_v[b], gsem[b])

            counts(a)

            @pl.when(g + 2 < NCHUNK)
            def _():
                stage_tok(g + 2, a)

            @pl.when(g >= 2)
            def _():
                pltpu.make_async_copy(
                    out_v[a], out_hbm.at[pl.ds(0, CHUNK)], osem[a]).wait()

            accumulate(g, a)
        return carry

    lax.fori_loop(0, NCHUNK // 2, loop_body, 0)
    for a in range(2):
        pltpu.make_async_copy(out_v[a], out_hbm.at[pl.ds(0, CHUNK)],
                              osem[a]).wait()


def kernel(header_tok, table):
    tok_flat = header_tok.reshape(-1)
    return _header_embed(tok_flat, table)
